# baseline (speedup 1.0000x reference)
import functools

import jax
import jax.numpy as jnp
from jax import lax
from jax.experimental import pallas as pl
from jax.experimental.pallas import tpu as pltpu
from jax.experimental.pallas import tpu_sc as plsc

B = 4096
C = 256
V = 512
ROWS = B * 16 * 8
NW = 32
K = 128
NCH = ROWS // (NW * K)


def _sc_body(table_hbm, idxg_hbm, out_hbm, idx_all, rows0, rows1,
             sg0, sg1, so0, so1):
    wid = lax.axis_index("s") * 2 + lax.axis_index("c")
    rows = (rows0, rows1)
    sg = (sg0, sg1)
    so = (so0, so1)
    pltpu.sync_copy(idxg_hbm.at[pl.ds(wid * NCH, NCH)], idx_all)

    @pl.loop(0, NCH, step=2)
    def _(ch0):
        for b in range(2):
            ch = ch0 + b
            base = (wid * NCH + ch) * K

            @pl.when(ch >= 2)
            def _():
                pltpu.make_async_copy(
                    rows[b], out_hbm.at[pl.ds(base, K)], so[b]).wait()

            pltpu.async_copy(
                table_hbm.at[idx_all.at[ch]], rows[b], sg[b]).wait()
            pltpu.async_copy(rows[b], out_hbm.at[pl.ds(base, K)], so[b])

    for b in range(2):
        pltpu.make_async_copy(
            rows[b], out_hbm.at[pl.ds(0, K)], so[b]).wait()


def kernel(indices, embedding):
    table2 = embedding.reshape(2 * V, 128)
    idxg = (indices.reshape(B, 16, 1, 4) * 2
            + jnp.arange(2, dtype=indices.dtype).reshape(1, 1, 2, 1))
    idxg = idxg.reshape(ROWS // 128, 128).astype(jnp.int32)

    mesh = plsc.VectorSubcoreMesh(core_axis_name="c", subcore_axis_name="s")
    run = pl.kernel(
        _sc_body,
        out_type=jax.ShapeDtypeStruct((ROWS, 128), jnp.float32),
        mesh=mesh,
        scratch_types=[
            pltpu.VMEM((NCH, K), jnp.int32),
            pltpu.VMEM((K, 128), jnp.float32),
            pltpu.VMEM((K, 128), jnp.float32),
            pltpu.SemaphoreType.DMA,
            pltpu.SemaphoreType.DMA,
            pltpu.SemaphoreType.DMA,
            pltpu.SemaphoreType.DMA,
        ],
    )
    out2 = run(table2, idxg)
    out6 = out2.reshape(B, 4, 4, 2, 4, 128)
    out5 = out6.transpose(0, 3, 5, 1, 2, 4)
    return out5.reshape(B, C, 4, 4, 4)


# speedup vs baseline: 1.9410x; 1.9410x over previous
import jax
import jax.numpy as jnp
from jax.experimental import pallas as pl

B = 4096
S = 64
C = 256
V = 512
G = 64
BLK = G * S


def _body(idx_ref, emb_ref, out_ref):
    idx_row = idx_ref[0]
    iota = jax.lax.broadcasted_iota(jnp.int16, (V, BLK), 0)
    oh = jnp.where(iota == idx_row, jnp.bfloat16(1), jnp.bfloat16(0))
    r = jax.lax.dot_general(
        oh, emb_ref[...],
        dimension_numbers=(((0,), (0,)), ((), ())),
        preferred_element_type=jnp.float32,
    )
    out_ref[:, 0:4, :] = r[:, :128].reshape(G * 16, 4, 128)
    out_ref[:, 4:8, :] = r[:, 128:].reshape(G * 16, 4, 128)


def kernel(indices, embedding):
    idx3 = indices.reshape(B // G, 1, BLK).astype(jnp.int16)
    emb16 = embedding.astype(jnp.bfloat16)
    out2 = pl.pallas_call(
        _body,
        grid=(B // G,),
        in_specs=[
            pl.BlockSpec((1, 1, BLK), lambda i: (i, 0, 0)),
            pl.BlockSpec((V, C), lambda i: (0, 0)),
        ],
        out_specs=pl.BlockSpec((G * 16, 8, 128), lambda i: (i, 0, 0)),
        out_shape=jax.ShapeDtypeStruct((B * 16, 8, 128), jnp.float32),
    )(idx3, emb16)
    out6 = out2.reshape(B, 4, 4, 2, 4, 128)

    out5 = out6.transpose(0, 3, 5, 1, 2, 4)
    return out5.reshape(B, C, 4, 4, 4)


# speedup vs baseline: 2.1081x; 1.0861x over previous
import jax
import jax.numpy as jnp
from jax.experimental import pallas as pl

B = 4096
S = 64
C = 256
V = 512
G = 128
BLK = G * S


def _body(idx_ref, emb_ref, out_ref):
    idx_row = idx_ref[0]
    iota = jax.lax.broadcasted_iota(jnp.int16, (V, BLK), 0)
    oh = jnp.where(iota == idx_row, jnp.bfloat16(1), jnp.bfloat16(0))
    r = jax.lax.dot_general(
        oh, emb_ref[...],
        dimension_numbers=(((0,), (0,)), ((), ())),
        preferred_element_type=jnp.float32,
    )
    out_ref[:, 0:4, :] = r[:, :128].reshape(G * 16, 4, 128)
    out_ref[:, 4:8, :] = r[:, 128:].reshape(G * 16, 4, 128)


def kernel(indices, embedding):
    idx3 = indices.reshape(B // G, 1, BLK).astype(jnp.int16)
    emb16 = embedding.astype(jnp.bfloat16)
    out2 = pl.pallas_call(
        _body,
        grid=(B // G,),
        in_specs=[
            pl.BlockSpec((1, 1, BLK), lambda i: (i, 0, 0)),
            pl.BlockSpec((V, C), lambda i: (0, 0)),
        ],
        out_specs=pl.BlockSpec((G * 16, 8, 128), lambda i: (i, 0, 0)),
        out_shape=jax.ShapeDtypeStruct((B * 16, 8, 128), jnp.float32),
    )(idx3, emb16)
    out6 = out2.reshape(B, 4, 4, 2, 4, 128)

    out5 = out6.transpose(0, 3, 5, 1, 2, 4)
    return out5.reshape(B, C, 4, 4, 4)


# speedup vs baseline: 2.1502x; 1.0200x over previous
import jax
import jax.numpy as jnp
from jax.experimental import pallas as pl

B = 4096
S = 64
C = 256
V = 512
G = 256
BLK = G * S


def _body(idx_ref, emb_ref, out_ref):
    idx_row = idx_ref[0]
    iota = jax.lax.broadcasted_iota(jnp.int16, (V, BLK), 0)
    oh = jnp.where(iota == idx_row, jnp.bfloat16(1), jnp.bfloat16(0))
    r = jax.lax.dot_general(
        oh, emb_ref[...],
        dimension_numbers=(((0,), (0,)), ((), ())),
        preferred_element_type=jnp.float32,
    )
    out_ref[:, 0:4, :] = r[:, :128].reshape(G * 16, 4, 128)
    out_ref[:, 4:8, :] = r[:, 128:].reshape(G * 16, 4, 128)


def kernel(indices, embedding):
    idx3 = indices.reshape(B // G, 1, BLK).astype(jnp.int16)
    emb16 = embedding.astype(jnp.bfloat16)
    out2 = pl.pallas_call(
        _body,
        grid=(B // G,),
        in_specs=[
            pl.BlockSpec((1, 1, BLK), lambda i: (i, 0, 0)),
            pl.BlockSpec((V, C), lambda i: (0, 0)),
        ],
        out_specs=pl.BlockSpec((G * 16, 8, 128), lambda i: (i, 0, 0)),
        out_shape=jax.ShapeDtypeStruct((B * 16, 8, 128), jnp.float32),
    )(idx3, emb16)
    out6 = out2.reshape(B, 4, 4, 2, 4, 128)

    out5 = out6.transpose(0, 3, 5, 1, 2, 4)
    return out5.reshape(B, C, 4, 4, 4)
